# Initial kernel scaffold; baseline (speedup 1.0000x reference)
import jax
import jax.numpy as jnp
from jax.experimental import pallas as pl


def kernel(x, edge_index, W1, b1, W2, b2):
    raise NotImplementedError("write your pallas kernel here")


# speedup vs baseline: 19.7415x; 19.7415x over previous
import jax
import jax.numpy as jnp
from jax import lax
from jax.experimental import pallas as pl
from jax.experimental.pallas import tpu as pltpu
from jax.experimental.pallas import tpu_sc as plsc

_N = 10000
_E = 320000
_DIN = 128
_DHID = 64
_DOUT = 16

_NC = 2
_NS = 16
_NW = _NC * _NS
_EPW = _E // _NW
_C = 128
_EPP = 10240
_PADE = _EPP - _EPW
_STEPS = _EPP // _C
_W = 128
_NPAD = 10240
_RPS = _NPAD // _NS

_mesh = plsc.VectorSubcoreMesh(core_axis_name="c", subcore_axis_name="s")


def _deg_body(dst_hbm, zeros_hbm, ones_hbm, deg_hbm, dstv, onesv, acc):
    c = lax.axis_index("c")
    s = lax.axis_index("s")
    wid = c * _NS + s
    slab = pl.ds(s * _RPS, _RPS)
    pltpu.sync_copy(zeros_hbm.at[slab], acc.at[slab])
    pltpu.sync_copy(ones_hbm, onesv)
    pltpu.sync_copy(dst_hbm.at[wid], dstv)
    plsc.subcore_barrier()
    @pl.loop(0, _STEPS)
    def _scatter(j):
        pltpu.sync_copy(onesv, acc.at[dstv.at[j]], add=True)
    plsc.subcore_barrier()
    pltpu.sync_copy(acc.at[slab], deg_hbm.at[c, slab])


def _prop_body(g_hbm, src_hbm, dst_hbm, zeros_hbm, out_hbm,
               srcv, dstv, rows, acc):
    c = lax.axis_index("c")
    s = lax.axis_index("s")
    wid = c * _NS + s
    slab = pl.ds(s * _RPS, _RPS)
    pltpu.sync_copy(zeros_hbm.at[slab], acc.at[slab])
    pltpu.sync_copy(src_hbm.at[wid], srcv)
    pltpu.sync_copy(dst_hbm.at[wid], dstv)
    plsc.subcore_barrier()
    @pl.loop(0, _STEPS)
    def _edge_chunk(j):
        pltpu.sync_copy(g_hbm.at[srcv.at[j]], rows)
        pltpu.sync_copy(rows, acc.at[dstv.at[j]], add=True)
    plsc.subcore_barrier()
    pltpu.sync_copy(acc.at[slab], out_hbm.at[c, slab])


def _sc_deg(dst3, zeros, ones):
    return pl.kernel(
        _deg_body,
        out_type=jax.ShapeDtypeStruct((_NC, _NPAD, _W), jnp.float32),
        mesh=_mesh,
        scratch_types=[
            pltpu.VMEM((_STEPS, _C), jnp.int32),
            pltpu.VMEM((_C, _W), jnp.float32),
            pltpu.VMEM_SHARED((_NPAD, _W), jnp.float32),
        ],
    )(dst3, zeros, ones)


def _sc_prop(g, src3, dst3, zeros):
    return pl.kernel(
        _prop_body,
        out_type=jax.ShapeDtypeStruct((_NC, _NPAD, _W), jnp.float32),
        mesh=_mesh,
        scratch_types=[
            pltpu.VMEM((_STEPS, _C), jnp.int32),
            pltpu.VMEM((_STEPS, _C), jnp.int32),
            pltpu.VMEM((_C, _W), jnp.float32),
            pltpu.VMEM_SHARED((_NPAD, _W), jnp.float32),
        ],
    )(g, src3, dst3, zeros)


_BLK = 1024


def _mm1_body(x_ref, w_ref, h_ref):
    h_ref[...] = jnp.dot(x_ref[...], w_ref[...],
                         preferred_element_type=jnp.float32)


def _g1_body(deg_ref, h_ref, g_ref):
    deg = 1.0 + deg_ref[0, :, 0] + deg_ref[1, :, 0]
    dinv = lax.rsqrt(deg)
    g = h_ref[...] * dinv[:, None]
    g_ref[...] = jnp.concatenate(
        [g, jnp.zeros((_BLK, _W - _DHID), jnp.float32)], axis=1)


def _mid_body(s_ref, deg_ref, h1_ref, b1_ref, w2_ref, h2_ref, g2_ref):
    deg = 1.0 + deg_ref[0, :, 0] + deg_ref[1, :, 0]
    dinv = lax.rsqrt(deg)
    dinv2 = 1.0 / deg
    a1 = dinv[:, None] * (s_ref[0, :, :_DHID] + s_ref[1, :, :_DHID]) \
        + dinv2[:, None] * h1_ref[...] + b1_ref[0, :]
    a1 = jnp.maximum(a1, 0.0)
    h2 = jnp.dot(a1, w2_ref[...], preferred_element_type=jnp.float32)
    h2_ref[...] = h2
    row = pl.program_id(0) * _BLK + lax.broadcasted_iota(
        jnp.int32, (_BLK, 1), 0)
    g2 = jnp.where(row < _N, h2 * dinv[:, None], 0.0)
    g2_ref[...] = jnp.concatenate(
        [g2, jnp.zeros((_BLK, _W - _DOUT), jnp.float32)], axis=1)


def _fin_body(s_ref, deg_ref, h2_ref, b2_ref, o_ref):
    deg = 1.0 + deg_ref[0, :, 0] + deg_ref[1, :, 0]
    dinv = lax.rsqrt(deg)
    dinv2 = 1.0 / deg
    o_ref[...] = dinv[:, None] * (s_ref[0, :, :_DOUT] + s_ref[1, :, :_DOUT]) \
        + dinv2[:, None] * h2_ref[...] + b2_ref[0, :]


def _row_spec(d):
    return pl.BlockSpec((_BLK, d), lambda i: (i, 0))


def _pair_spec(d):
    return pl.BlockSpec((_NC, _BLK, d), lambda i: (0, i, 0))


def _full_spec(a, b):
    return pl.BlockSpec((a, b), lambda i: (0, 0))


def kernel(x, edge_index, W1, b1, W2, b2):
    pad_idx = jnp.broadcast_to(
        _N + (jnp.arange(_PADE, dtype=jnp.int32) % (_NPAD - _N)),
        (_NW, _PADE))
    src3 = jnp.concatenate(
        [edge_index[0].reshape(_NW, _EPW), pad_idx], axis=1
    ).reshape(_NW, _STEPS, _C)
    dst3 = jnp.concatenate(
        [edge_index[1].reshape(_NW, _EPW), pad_idx], axis=1
    ).reshape(_NW, _STEPS, _C)
    xp = jnp.pad(x, ((0, _NPAD - _N), (0, 0)))
    zeros = jnp.zeros((_NPAD, _W), jnp.float32)
    ones = jnp.ones((_C, _W), jnp.float32)
    b1r = b1.reshape(1, _DHID)
    b2r = b2.reshape(1, _DOUT)

    grid = _NPAD // _BLK

    deg2 = _sc_deg(dst3, zeros, ones)

    h1 = pl.pallas_call(
        _mm1_body,
        grid=(grid,),
        in_specs=[_row_spec(_DIN), _full_spec(_DIN, _DHID)],
        out_specs=_row_spec(_DHID),
        out_shape=jax.ShapeDtypeStruct((_NPAD, _DHID), jnp.float32),
    )(xp, W1)

    g1 = pl.pallas_call(
        _g1_body,
        grid=(grid,),
        in_specs=[_pair_spec(_W), _row_spec(_DHID)],
        out_specs=_row_spec(_W),
        out_shape=jax.ShapeDtypeStruct((_NPAD, _W), jnp.float32),
    )(deg2, h1)

    s1 = _sc_prop(g1, src3, dst3, zeros)

    h2, g2 = pl.pallas_call(
        _mid_body,
        grid=(grid,),
        in_specs=[_pair_spec(_W), _pair_spec(_W), _row_spec(_DHID),
                  _full_spec(1, _DHID), _full_spec(_DHID, _DOUT)],
        out_specs=[_row_spec(_DOUT), _row_spec(_W)],
        out_shape=[jax.ShapeDtypeStruct((_NPAD, _DOUT), jnp.float32),
                   jax.ShapeDtypeStruct((_NPAD, _W), jnp.float32)],
    )(s1, deg2, h1, b1r, W2)

    s2 = _sc_prop(g2, src3, dst3, zeros)

    out = pl.pallas_call(
        _fin_body,
        grid=(grid,),
        in_specs=[_pair_spec(_W), _pair_spec(_W), _row_spec(_DOUT),
                  _full_spec(1, _DOUT)],
        out_specs=_row_spec(_DOUT),
        out_shape=jax.ShapeDtypeStruct((_NPAD, _DOUT), jnp.float32),
    )(s2, deg2, h2, b2r)

    return out[:_N]


# speedup vs baseline: 22.5086x; 1.1402x over previous
import jax
import jax.numpy as jnp
from jax import lax
from jax.experimental import pallas as pl
from jax.experimental.pallas import tpu as pltpu
from jax.experimental.pallas import tpu_sc as plsc

_N = 10000
_E = 320000
_DIN = 128
_DHID = 64
_DOUT = 16

_NC = 2
_NS = 16
_NW = _NC * _NS
_EPW = _E // _NW
_C = 128
_EPP = 10240
_PADE = _EPP - _EPW
_STEPS = _EPP // _C
_W = 128
_NPAD = 10240
_RPS = _NPAD // _NS

_mesh = plsc.VectorSubcoreMesh(core_axis_name="c", subcore_axis_name="s")


def _deg_body(dst_hbm, zeros_hbm, ones_hbm, deg_hbm, dstv, onesv, acc, sd):
    c = lax.axis_index("c")
    s = lax.axis_index("s")
    wid = c * _NS + s
    slab = pl.ds(s * _RPS, _RPS)
    pltpu.sync_copy(zeros_hbm.at[slab], acc.at[slab])
    pltpu.sync_copy(ones_hbm, onesv)
    pltpu.sync_copy(dst_hbm.at[wid], dstv)
    plsc.subcore_barrier()
    @pl.loop(0, _STEPS)
    def _fire(j):
        pltpu.async_copy(onesv, acc.at[dstv.at[j]], sd, add=True)
    @pl.loop(0, _STEPS)
    def _drain(j):
        pltpu.make_async_copy(onesv, acc.at[dstv.at[j]], sd).wait()
    plsc.subcore_barrier()
    pltpu.sync_copy(acc.at[slab], deg_hbm.at[c, slab])


def _prop_body(g_hbm, src_hbm, dst_hbm, zeros_hbm, out_hbm,
               srcv2, dstv, rows0, rows1, acc,
               sg0, sg1, ss0, ss1, si0, si1):
    c = lax.axis_index("c")
    s = lax.axis_index("s")
    wid = c * _NS + s
    slab = pl.ds(s * _RPS, _RPS)
    pltpu.sync_copy(zeros_hbm.at[slab], acc.at[slab])
    pltpu.sync_copy(dst_hbm.at[wid], dstv)
    pltpu.sync_copy(src_hbm.at[wid, 0], srcv2.at[0])
    pltpu.sync_copy(src_hbm.at[wid, 1], srcv2.at[1])
    plsc.subcore_barrier()
    pltpu.async_copy(g_hbm.at[srcv2.at[0]], rows0, sg0)
    pltpu.async_copy(g_hbm.at[srcv2.at[1]], rows1, sg1)

    @pl.loop(0, _STEPS, step=2)
    def _pipe(j):
        pltpu.make_async_copy(g_hbm.at[srcv2.at[0]], rows0, sg0).wait()
        pltpu.async_copy(rows0, acc.at[dstv.at[j]], ss0, add=True)
        @pl.when(j + 2 < _STEPS)
        def _():
            pltpu.async_copy(src_hbm.at[wid, j + 2], srcv2.at[0], si0)
        pltpu.make_async_copy(g_hbm.at[srcv2.at[1]], rows1, sg1).wait()
        pltpu.async_copy(rows1, acc.at[dstv.at[j + 1]], ss1, add=True)
        @pl.when(j + 3 < _STEPS)
        def _():
            pltpu.async_copy(src_hbm.at[wid, j + 3], srcv2.at[1], si1)
        pltpu.make_async_copy(rows0, acc.at[dstv.at[j]], ss0).wait()
        @pl.when(j + 2 < _STEPS)
        def _():
            pltpu.make_async_copy(
                src_hbm.at[wid, j + 2], srcv2.at[0], si0).wait()
            pltpu.async_copy(g_hbm.at[srcv2.at[0]], rows0, sg0)
        pltpu.make_async_copy(rows1, acc.at[dstv.at[j + 1]], ss1).wait()
        @pl.when(j + 3 < _STEPS)
        def _():
            pltpu.make_async_copy(
                src_hbm.at[wid, j + 3], srcv2.at[1], si1).wait()
            pltpu.async_copy(g_hbm.at[srcv2.at[1]], rows1, sg1)

    plsc.subcore_barrier()
    pltpu.sync_copy(acc.at[slab], out_hbm.at[c, slab])


def _sc_deg(dst3, zeros, ones):
    return pl.kernel(
        _deg_body,
        out_type=jax.ShapeDtypeStruct((_NC, _NPAD, _W), jnp.float32),
        mesh=_mesh,
        scratch_types=[
            pltpu.VMEM((_STEPS, _C), jnp.int32),
            pltpu.VMEM((_C, _W), jnp.float32),
            pltpu.VMEM_SHARED((_NPAD, _W), jnp.float32),
            pltpu.SemaphoreType.DMA,
        ],
    )(dst3, zeros, ones)


def _sc_prop(g, src3, dst3, zeros):
    return pl.kernel(
        _prop_body,
        out_type=jax.ShapeDtypeStruct((_NC, _NPAD, _W), jnp.float32),
        mesh=_mesh,
        scratch_types=[
            pltpu.VMEM((2, _C), jnp.int32),
            pltpu.VMEM((_STEPS, _C), jnp.int32),
            pltpu.VMEM((_C, _W), jnp.float32),
            pltpu.VMEM((_C, _W), jnp.float32),
            pltpu.VMEM_SHARED((_NPAD, _W), jnp.float32),
            pltpu.SemaphoreType.DMA,
            pltpu.SemaphoreType.DMA,
            pltpu.SemaphoreType.DMA,
            pltpu.SemaphoreType.DMA,
            pltpu.SemaphoreType.DMA,
            pltpu.SemaphoreType.DMA,
        ],
    )(g, src3, dst3, zeros)


_BLK = 1024


def _mm1_body(x_ref, w_ref, h_ref):
    h_ref[...] = jnp.dot(x_ref[...], w_ref[...],
                         preferred_element_type=jnp.float32)


def _g1_body(deg_ref, h_ref, g_ref):
    deg = 1.0 + deg_ref[0, :, 0] + deg_ref[1, :, 0]
    dinv = lax.rsqrt(deg)
    g = h_ref[...] * dinv[:, None]
    g_ref[...] = jnp.concatenate(
        [g, jnp.zeros((_BLK, _W - _DHID), jnp.float32)], axis=1)


def _mid_body(s_ref, deg_ref, h1_ref, b1_ref, w2_ref, h2_ref, g2_ref):
    deg = 1.0 + deg_ref[0, :, 0] + deg_ref[1, :, 0]
    dinv = lax.rsqrt(deg)
    dinv2 = 1.0 / deg
    a1 = dinv[:, None] * (s_ref[0, :, :_DHID] + s_ref[1, :, :_DHID]) \
        + dinv2[:, None] * h1_ref[...] + b1_ref[0, :]
    a1 = jnp.maximum(a1, 0.0)
    h2 = jnp.dot(a1, w2_ref[...], preferred_element_type=jnp.float32)
    h2_ref[...] = h2
    row = pl.program_id(0) * _BLK + lax.broadcasted_iota(
        jnp.int32, (_BLK, 1), 0)
    g2 = jnp.where(row < _N, h2 * dinv[:, None], 0.0)
    g2_ref[...] = jnp.concatenate(
        [g2, jnp.zeros((_BLK, _W - _DOUT), jnp.float32)], axis=1)


def _fin_body(s_ref, deg_ref, h2_ref, b2_ref, o_ref):
    deg = 1.0 + deg_ref[0, :, 0] + deg_ref[1, :, 0]
    dinv = lax.rsqrt(deg)
    dinv2 = 1.0 / deg
    o_ref[...] = dinv[:, None] * (s_ref[0, :, :_DOUT] + s_ref[1, :, :_DOUT]) \
        + dinv2[:, None] * h2_ref[...] + b2_ref[0, :]


def _row_spec(d):
    return pl.BlockSpec((_BLK, d), lambda i: (i, 0))


def _pair_spec(d):
    return pl.BlockSpec((_NC, _BLK, d), lambda i: (0, i, 0))


def _full_spec(a, b):
    return pl.BlockSpec((a, b), lambda i: (0, 0))


def kernel(x, edge_index, W1, b1, W2, b2):
    pad_idx = jnp.broadcast_to(
        _N + (jnp.arange(_PADE, dtype=jnp.int32) % (_NPAD - _N)),
        (_NW, _PADE))
    src3 = jnp.concatenate(
        [edge_index[0].reshape(_NW, _EPW), pad_idx], axis=1
    ).reshape(_NW, _STEPS, _C)
    dst3 = jnp.concatenate(
        [edge_index[1].reshape(_NW, _EPW), pad_idx], axis=1
    ).reshape(_NW, _STEPS, _C)
    xp = jnp.pad(x, ((0, _NPAD - _N), (0, 0)))
    zeros = jnp.zeros((_NPAD, _W), jnp.float32)
    ones = jnp.ones((_C, _W), jnp.float32)
    b1r = b1.reshape(1, _DHID)
    b2r = b2.reshape(1, _DOUT)

    grid = _NPAD // _BLK

    deg2 = _sc_deg(dst3, zeros, ones)

    h1 = pl.pallas_call(
        _mm1_body,
        grid=(grid,),
        in_specs=[_row_spec(_DIN), _full_spec(_DIN, _DHID)],
        out_specs=_row_spec(_DHID),
        out_shape=jax.ShapeDtypeStruct((_NPAD, _DHID), jnp.float32),
    )(xp, W1)

    g1 = pl.pallas_call(
        _g1_body,
        grid=(grid,),
        in_specs=[_pair_spec(_W), _row_spec(_DHID)],
        out_specs=_row_spec(_W),
        out_shape=jax.ShapeDtypeStruct((_NPAD, _W), jnp.float32),
    )(deg2, h1)

    s1 = _sc_prop(g1, src3, dst3, zeros)

    h2, g2 = pl.pallas_call(
        _mid_body,
        grid=(grid,),
        in_specs=[_pair_spec(_W), _pair_spec(_W), _row_spec(_DHID),
                  _full_spec(1, _DHID), _full_spec(_DHID, _DOUT)],
        out_specs=[_row_spec(_DOUT), _row_spec(_W)],
        out_shape=[jax.ShapeDtypeStruct((_NPAD, _DOUT), jnp.float32),
                   jax.ShapeDtypeStruct((_NPAD, _W), jnp.float32)],
    )(s1, deg2, h1, b1r, W2)

    s2 = _sc_prop(g2, src3, dst3, zeros)

    out = pl.pallas_call(
        _fin_body,
        grid=(grid,),
        in_specs=[_pair_spec(_W), _pair_spec(_W), _row_spec(_DOUT),
                  _full_spec(1, _DOUT)],
        out_specs=_row_spec(_DOUT),
        out_shape=jax.ShapeDtypeStruct((_NPAD, _DOUT), jnp.float32),
    )(s2, deg2, h2, b2r)

    return out[:_N]


# speedup vs baseline: 26.2346x; 1.1655x over previous
import jax
import jax.numpy as jnp
from jax import lax
from jax.experimental import pallas as pl
from jax.experimental.pallas import tpu as pltpu
from jax.experimental.pallas import tpu_sc as plsc

_N = 10000
_E = 320000
_DIN = 128
_DHID = 64
_DOUT = 16

_NC = 2
_NS = 16
_NW = _NC * _NS
_EPW = _E // _NW
_C = 80
_EPP = 10240
_PADE = _EPP - _EPW
_STEPS = _EPP // _C
_NBUF = 4
_W = 128
_NPAD = 10240
_RPS = _NPAD // _NS

_mesh = plsc.VectorSubcoreMesh(core_axis_name="c", subcore_axis_name="s")


def _deg_body(dst_hbm, zeros_hbm, ones_hbm, deg_hbm, dstv, onesv, acc, sd):
    c = lax.axis_index("c")
    s = lax.axis_index("s")
    wid = c * _NS + s
    slab = pl.ds(s * _RPS, _RPS)
    pltpu.sync_copy(zeros_hbm.at[slab], acc.at[slab])
    pltpu.sync_copy(ones_hbm, onesv)
    pltpu.sync_copy(dst_hbm.at[wid], dstv)
    plsc.subcore_barrier()
    @pl.loop(0, _STEPS)
    def _fire(j):
        pltpu.async_copy(onesv, acc.at[dstv.at[j]], sd, add=True)
    @pl.loop(0, _STEPS)
    def _drain(j):
        pltpu.make_async_copy(onesv, acc.at[dstv.at[j]], sd).wait()
    plsc.subcore_barrier()
    pltpu.sync_copy(acc.at[slab], deg_hbm.at[c, slab])


def _prop_body(g_hbm, src_hbm, dst_hbm, zeros_hbm, out_hbm,
               srcv, dstv, r0, r1, r2, r3, acc, *sems):
    rows = [r0, r1, r2, r3]
    sg, ss, si, sd = (sems[0:4], sems[4:8], sems[8:12], sems[12:16])
    c = lax.axis_index("c")
    s = lax.axis_index("s")
    wid = c * _NS + s
    slab = pl.ds(s * _RPS, _RPS)
    pltpu.sync_copy(zeros_hbm.at[slab], acc.at[slab])
    plsc.subcore_barrier()
    for b in range(_NBUF):
        pltpu.async_copy(src_hbm.at[wid, b], srcv.at[b], si[b])
        pltpu.async_copy(dst_hbm.at[wid, b], dstv.at[b], sd[b])
    for b in range(_NBUF):
        pltpu.make_async_copy(src_hbm.at[wid, b], srcv.at[b], si[b]).wait()
        pltpu.async_copy(g_hbm.at[srcv.at[b]], rows[b], sg[b])

    @pl.loop(0, _STEPS, step=_NBUF)
    def _pipe(j):
        for b in range(_NBUF):
            k = j + b
            pltpu.make_async_copy(
                dst_hbm.at[wid, b], dstv.at[b], sd[b]).wait()
            pltpu.make_async_copy(g_hbm.at[srcv.at[b]], rows[b], sg[b]).wait()
            pltpu.async_copy(rows[b], acc.at[dstv.at[b]], ss[b], add=True)
            @pl.when(k + _NBUF < _STEPS)
            def _(b=b, k=k):
                pltpu.async_copy(src_hbm.at[wid, k + _NBUF], srcv.at[b],
                                 si[b])
        for b in range(_NBUF):
            k = j + b
            pltpu.make_async_copy(rows[b], acc.at[dstv.at[b]], ss[b]).wait()
            @pl.when(k + _NBUF < _STEPS)
            def _(b=b, k=k):
                pltpu.async_copy(dst_hbm.at[wid, k + _NBUF], dstv.at[b],
                                 sd[b])
                pltpu.make_async_copy(src_hbm.at[wid, k + _NBUF],
                                      srcv.at[b], si[b]).wait()
                pltpu.async_copy(g_hbm.at[srcv.at[b]], rows[b], sg[b])

    plsc.subcore_barrier()
    pltpu.sync_copy(acc.at[slab], out_hbm.at[c, slab])


def _sc_deg(dst3, zeros, ones):
    return pl.kernel(
        _deg_body,
        out_type=jax.ShapeDtypeStruct((_NC, _NPAD, _W), jnp.float32),
        mesh=_mesh,
        scratch_types=[
            pltpu.VMEM((_STEPS, _C), jnp.int32),
            pltpu.VMEM((_C, _W), jnp.float32),
            pltpu.VMEM_SHARED((_NPAD, _W), jnp.float32),
            pltpu.SemaphoreType.DMA,
        ],
    )(dst3, zeros, ones)


def _sc_prop(g, src3, dst3, zeros):
    return pl.kernel(
        _prop_body,
        out_type=jax.ShapeDtypeStruct((_NC, _NPAD, _W), jnp.float32),
        mesh=_mesh,
        scratch_types=[
            pltpu.VMEM((_NBUF, _C), jnp.int32),
            pltpu.VMEM((_NBUF, _C), jnp.int32),
            pltpu.VMEM((_C, _W), jnp.float32),
            pltpu.VMEM((_C, _W), jnp.float32),
            pltpu.VMEM((_C, _W), jnp.float32),
            pltpu.VMEM((_C, _W), jnp.float32),
            pltpu.VMEM_SHARED((_NPAD, _W), jnp.float32),
        ] + [pltpu.SemaphoreType.DMA] * 16,
    )(g, src3, dst3, zeros)


_BLK = 1024


def _mm1_body(x_ref, w_ref, h_ref):
    h_ref[...] = jnp.dot(x_ref[...], w_ref[...],
                         preferred_element_type=jnp.float32)


def _g1_body(deg_ref, h_ref, g_ref):
    deg = 1.0 + deg_ref[0, :, 0] + deg_ref[1, :, 0]
    dinv = lax.rsqrt(deg)
    g = h_ref[...] * dinv[:, None]
    g_ref[...] = jnp.concatenate(
        [g, jnp.zeros((_BLK, _W - _DHID), jnp.float32)], axis=1)


def _mid_body(s_ref, deg_ref, h1_ref, b1_ref, w2_ref, h2_ref, g2_ref):
    deg = 1.0 + deg_ref[0, :, 0] + deg_ref[1, :, 0]
    dinv = lax.rsqrt(deg)
    dinv2 = 1.0 / deg
    a1 = dinv[:, None] * (s_ref[0, :, :_DHID] + s_ref[1, :, :_DHID]) \
        + dinv2[:, None] * h1_ref[...] + b1_ref[0, :]
    a1 = jnp.maximum(a1, 0.0)
    h2 = jnp.dot(a1, w2_ref[...], preferred_element_type=jnp.float32)
    h2_ref[...] = h2
    row = pl.program_id(0) * _BLK + lax.broadcasted_iota(
        jnp.int32, (_BLK, 1), 0)
    g2 = jnp.where(row < _N, h2 * dinv[:, None], 0.0)
    g2_ref[...] = jnp.concatenate(
        [g2, jnp.zeros((_BLK, _W - _DOUT), jnp.float32)], axis=1)


def _fin_body(s_ref, deg_ref, h2_ref, b2_ref, o_ref):
    deg = 1.0 + deg_ref[0, :, 0] + deg_ref[1, :, 0]
    dinv = lax.rsqrt(deg)
    dinv2 = 1.0 / deg
    o_ref[...] = dinv[:, None] * (s_ref[0, :, :_DOUT] + s_ref[1, :, :_DOUT]) \
        + dinv2[:, None] * h2_ref[...] + b2_ref[0, :]


def _row_spec(d):
    return pl.BlockSpec((_BLK, d), lambda i: (i, 0))


def _pair_spec(d):
    return pl.BlockSpec((_NC, _BLK, d), lambda i: (0, i, 0))


def _full_spec(a, b):
    return pl.BlockSpec((a, b), lambda i: (0, 0))


def kernel(x, edge_index, W1, b1, W2, b2):
    pad_idx = jnp.broadcast_to(
        _N + (jnp.arange(_PADE, dtype=jnp.int32) % (_NPAD - _N)),
        (_NW, _PADE))
    src3 = jnp.concatenate(
        [edge_index[0].reshape(_NW, _EPW), pad_idx], axis=1
    ).reshape(_NW, _STEPS, _C)
    dst3 = jnp.concatenate(
        [edge_index[1].reshape(_NW, _EPW), pad_idx], axis=1
    ).reshape(_NW, _STEPS, _C)
    xp = jnp.pad(x, ((0, _NPAD - _N), (0, 0)))
    zeros = jnp.zeros((_NPAD, _W), jnp.float32)
    ones = jnp.ones((_C, _W), jnp.float32)
    b1r = b1.reshape(1, _DHID)
    b2r = b2.reshape(1, _DOUT)

    grid = _NPAD // _BLK

    deg2 = _sc_deg(dst3, zeros, ones)

    h1 = pl.pallas_call(
        _mm1_body,
        grid=(grid,),
        in_specs=[_row_spec(_DIN), _full_spec(_DIN, _DHID)],
        out_specs=_row_spec(_DHID),
        out_shape=jax.ShapeDtypeStruct((_NPAD, _DHID), jnp.float32),
    )(xp, W1)

    g1 = pl.pallas_call(
        _g1_body,
        grid=(grid,),
        in_specs=[_pair_spec(_W), _row_spec(_DHID)],
        out_specs=_row_spec(_W),
        out_shape=jax.ShapeDtypeStruct((_NPAD, _W), jnp.float32),
    )(deg2, h1)

    s1 = _sc_prop(g1, src3, dst3, zeros)

    h2, g2 = pl.pallas_call(
        _mid_body,
        grid=(grid,),
        in_specs=[_pair_spec(_W), _pair_spec(_W), _row_spec(_DHID),
                  _full_spec(1, _DHID), _full_spec(_DHID, _DOUT)],
        out_specs=[_row_spec(_DOUT), _row_spec(_W)],
        out_shape=[jax.ShapeDtypeStruct((_NPAD, _DOUT), jnp.float32),
                   jax.ShapeDtypeStruct((_NPAD, _W), jnp.float32)],
    )(s1, deg2, h1, b1r, W2)

    s2 = _sc_prop(g2, src3, dst3, zeros)

    out = pl.pallas_call(
        _fin_body,
        grid=(grid,),
        in_specs=[_pair_spec(_W), _pair_spec(_W), _row_spec(_DOUT),
                  _full_spec(1, _DOUT)],
        out_specs=_row_spec(_DOUT),
        out_shape=jax.ShapeDtypeStruct((_NPAD, _DOUT), jnp.float32),
    )(s2, deg2, h2, b2r)

    return out[:_N]
